# baseline (speedup 1.0000x reference)
import jax
import jax.numpy as jnp
from jax import lax
from jax.experimental import pallas as pl
from jax.experimental.pallas import tpu as pltpu
from jax.experimental.pallas import tpu_sc as plsc

B = 4096
L = 200
EMBED = 64
_HALF = L // 2

_info = plsc.get_sparse_core_info()
_NC, _NS = _info.num_cores, _info.num_subcores
_NW = _NC * _NS
_SEQ_PER_W = B // _NW


def _embed_body(x_hbm, tok_hbm, pos_hbm, out_hbm, idx_v, pos_v, rows_v, sem):
    wid = lax.axis_index("s") * _NC + lax.axis_index("c")
    base = wid * _SEQ_PER_W
    pltpu.sync_copy(x_hbm.at[pl.ds(base, _SEQ_PER_W)], idx_v)
    pltpu.sync_copy(pos_hbm.at[pl.ds(0, L)], pos_v)

    def seq_body(s, carry):
        pltpu.async_copy(tok_hbm.at[idx_v.at[s, 0]],
                         rows_v.at[pl.ds(0, _HALF)], sem).wait()
        pltpu.async_copy(tok_hbm.at[idx_v.at[s, 1]],
                         rows_v.at[pl.ds(_HALF, _HALF)], sem).wait()

        def add_row(r, c):
            for d in range(EMBED // 16):
                sl = pl.ds(d * 16, 16)
                rows_v[r, sl] = rows_v[r, sl] + pos_v[r, sl]
            return c

        lax.fori_loop(0, L, add_row, 0)
        pltpu.sync_copy(rows_v, out_hbm.at[base + s])
        return carry

    lax.fori_loop(0, _SEQ_PER_W, seq_body, 0)


def kernel(x, tok_table, pos_table):
    x3 = x.astype(jnp.int32).reshape(B, 2, _HALF)
    mesh = plsc.VectorSubcoreMesh(core_axis_name="c", subcore_axis_name="s")
    f = pl.kernel(
        _embed_body,
        mesh=mesh,
        out_type=jax.ShapeDtypeStruct((B, L, EMBED), jnp.float32),
        scratch_types=[
            pltpu.VMEM((_SEQ_PER_W, 2, _HALF), jnp.int32),
            pltpu.VMEM((L, EMBED), jnp.float32),
            pltpu.VMEM((L, EMBED), jnp.float32),
            pltpu.SemaphoreType.DMA,
        ],
        compiler_params=pltpu.CompilerParams(use_tc_tiling_on_sc=False),
    )
    return f(x3, tok_table, pos_table)


# speedup vs baseline: 1.2143x; 1.2143x over previous
import jax
import jax.numpy as jnp
from jax import lax
from jax.experimental import pallas as pl
from jax.experimental.pallas import tpu as pltpu
from jax.experimental.pallas import tpu_sc as plsc

B = 4096
L = 200
EMBED = 64
_HALF = L // 2
_NBUF = 4
_LOOK = 2

_info = plsc.get_sparse_core_info()
_NC, _NS = _info.num_cores, _info.num_subcores
_NW = _NC * _NS
_SEQ_PER_W = B // _NW


def _embed_body(x_hbm, tok_hbm, pos_hbm, out_hbm, idx_v, pos_v, rows_v,
                gsem, osem):
    wid = lax.axis_index("s") * _NC + lax.axis_index("c")
    base = wid * _SEQ_PER_W
    pltpu.sync_copy(x_hbm.at[pl.ds(base, _SEQ_PER_W)], idx_v)
    pltpu.sync_copy(pos_hbm.at[pl.ds(0, L)], pos_v)

    def issue_gather(s, b):
        pltpu.async_copy(tok_hbm.at[idx_v.at[s, 0]],
                         rows_v.at[b, pl.ds(0, _HALF)], gsem.at[b])
        pltpu.async_copy(tok_hbm.at[idx_v.at[s, 1]],
                         rows_v.at[b, pl.ds(_HALF, _HALF)], gsem.at[b])

    def wait_gather(s, b):
        pltpu.make_async_copy(tok_hbm.at[idx_v.at[s, 0]],
                              rows_v.at[b, pl.ds(0, _HALF)], gsem.at[b]).wait()
        pltpu.make_async_copy(tok_hbm.at[idx_v.at[s, 1]],
                              rows_v.at[b, pl.ds(_HALF, _HALF)],
                              gsem.at[b]).wait()

    def issue_out(s, b):
        pltpu.async_copy(rows_v.at[b], out_hbm.at[base + s], osem.at[b])

    def wait_out(b):
        pltpu.make_async_copy(rows_v.at[b], out_hbm.at[base], osem.at[b]).wait()

    def compute(b):
        def add_rows(r4, c):
            for u in range(4):
                r = r4 * 4 + u
                for d in range(EMBED // 16):
                    sl = pl.ds(d * 16, 16)
                    rows_v[b, r, sl] = rows_v[b, r, sl] + pos_v[r, sl]
            return c
        lax.fori_loop(0, L // 4, add_rows, 0)

    def step(s, b, skip_out_wait):
        nb = (b + _LOOK) % _NBUF
        if skip_out_wait:
            issue_gather(s + _LOOK, nb)
        else:
            @pl.when(s + _LOOK < _SEQ_PER_W)
            def _():
                wait_out(nb)
                issue_gather(s + _LOOK, nb)
        wait_gather(s, b)
        compute(b)
        issue_out(s, b)

    issue_gather(0, 0)
    issue_gather(1, 1)
    for b in range(_NBUF):
        step(b, b, skip_out_wait=(b < _LOOK))

    def group(g, c):
        for b in range(_NBUF):
            step(g * _NBUF + b, b, skip_out_wait=False)
        return c

    lax.fori_loop(1, _SEQ_PER_W // _NBUF, group, 0)

    for b in range(_NBUF):
        wait_out(b)


def kernel(x, tok_table, pos_table):
    x3 = x.astype(jnp.int32).reshape(B, 2, _HALF)
    mesh = plsc.VectorSubcoreMesh(core_axis_name="c", subcore_axis_name="s")
    f = pl.kernel(
        _embed_body,
        mesh=mesh,
        out_type=jax.ShapeDtypeStruct((B, L, EMBED), jnp.float32),
        scratch_types=[
            pltpu.VMEM((_SEQ_PER_W, 2, _HALF), jnp.int32),
            pltpu.VMEM((L, EMBED), jnp.float32),
            pltpu.VMEM((_NBUF, L, EMBED), jnp.float32),
            pltpu.SemaphoreType.DMA((_NBUF,)),
            pltpu.SemaphoreType.DMA((_NBUF,)),
        ],
        compiler_params=pltpu.CompilerParams(use_tc_tiling_on_sc=False),
    )
    return f(x3, tok_table, pos_table)
